# baseline (speedup 1.0000x reference)
import functools

import jax
import jax.numpy as jnp
from jax.experimental import pallas as pl

_NQ = 64
_BLK = 16384
_NCLS = 4096
_LANES = 128
_NEG = float("-inf")


def _stream_kernel(n, q_ref, c_ref, v1_ref, i1_ref, v2_ref, i2_ref, v3m_ref):
    j = pl.program_id(0)

    @pl.when(j == 0)
    def _init():
        v1_ref[...] = jnp.full((_NQ, _NCLS), _NEG, jnp.float32)
        v2_ref[...] = jnp.full((_NQ, _NCLS), _NEG, jnp.float32)
        v3m_ref[...] = jnp.full((_NQ, _LANES), _NEG, jnp.float32)

    scores = jax.lax.dot_general(
        q_ref[...].astype(jnp.bfloat16), c_ref[...],
        dimension_numbers=(((1,), (0,)), ((), ())),
        preferred_element_type=jnp.float32,
    )
    lane = jax.lax.broadcasted_iota(jnp.int32, (_NQ, _LANES), 1)
    base = j * _BLK
    quart = _BLK // 4
    v3m = v3m_ref[...]

    def _pair(xv, xi, yv, yi):
        c = yv > xv
        return (jnp.where(c, yv, xv), jnp.where(c, yi, xi),
                jnp.where(c, xv, yv), jnp.where(c, xi, yi), c)

    for g in range(_NCLS // _LANES):
        sl = slice(g * _LANES, (g + 1) * _LANES)
        col = g * _LANES
        xs, xis = [], []
        for qq in range(4):
            mi = lane + (base + qq * quart + col)
            v = scores[:, qq * quart + col:qq * quart + col + _LANES]
            xs.append(jnp.where(mi < n, v, _NEG))
            xis.append(mi)

        h1, h1i, l1, l1i, _ = _pair(xs[0], xis[0], xs[1], xis[1])
        h2, h2i, l2, l2i, _ = _pair(xs[2], xis[2], xs[3], xis[3])
        c = h2 > h1
        hi = jnp.where(c, h2, h1)
        hi_i = jnp.where(c, h2i, h1i)
        m1 = jnp.minimum(h1, h2)
        m1i = jnp.where(c, h1i, h2i)
        s = jnp.where(c, l2, l1)
        s_i = jnp.where(c, l2i, l1i)
        cq = s > m1
        lo = jnp.where(cq, s, m1)
        lo_i = jnp.where(cq, s_i, m1i)
        rest = jnp.maximum(jnp.where(cq, m1, s), jnp.where(c, l1, l2))

        v1 = v1_ref[:, sl]
        i1 = i1_ref[:, sl]
        v2 = v2_ref[:, sl]
        i2 = i2_ref[:, sl]

        c1 = hi > v1
        v1_ref[:, sl] = jnp.where(c1, hi, v1)
        i1_ref[:, sl] = jnp.where(c1, hi_i, i1)
        m2 = jnp.minimum(v1, hi)
        m2i = jnp.where(c1, i1, hi_i)
        s2 = jnp.where(c1, lo, v2)
        s2i = jnp.where(c1, lo_i, i2)
        c2 = s2 > m2
        v2_ref[:, sl] = jnp.where(c2, s2, m2)
        i2_ref[:, sl] = jnp.where(c2, s2i, m2i)

        t1 = jnp.where(c2, m2, s2)
        t2 = jnp.where(c1, v2, lo)
        v3m = jnp.maximum(v3m, jnp.maximum(jnp.maximum(t1, t2), rest))

    v3m_ref[...] = v3m


_IMAX = 2**31 - 1


def _extract_kernel(v1_ref, i1_ref, v2_ref, i2_ref, v3_ref,
                    vals_ref, idx_ref, flag_ref):
    v1 = v1_ref[...]
    v2 = v2_ref[...]
    i1 = i1_ref[...]
    i2 = i2_ref[...]
    vals_ref[...] = jnp.zeros((_NQ, _LANES), jnp.float32)
    idx_ref[...] = jnp.zeros((_NQ, _LANES), jnp.int32)
    cm = None
    for t in range(10):
        m = jnp.maximum(jnp.max(v1, axis=1, keepdims=True),
                        jnp.max(v2, axis=1, keepdims=True))
        eq1 = v1 == m
        eq2 = v2 == m
        ix = jnp.minimum(
            jnp.min(jnp.where(eq1, i1, _IMAX), axis=1, keepdims=True),
            jnp.min(jnp.where(eq2, i2, _IMAX), axis=1, keepdims=True))
        v1 = jnp.where(eq1 & (i1 == ix), _NEG, v1)
        v2 = jnp.where(eq2 & (i2 == ix), _NEG, v2)
        vals_ref[:, t:t + 1] = m
        idx_ref[:, t:t + 1] = ix
        cm = m
    f1 = jnp.max(jnp.where(v3_ref[...] >= cm, 1.0, 0.0), axis=1, keepdims=True)
    flag_ref[...] = jnp.zeros((_NQ, _LANES), jnp.float32)
    flag_ref[:, 0:1] = f1


def _mm_kernel(q_ref, c_ref, o_ref):
    o_ref[...] = jax.lax.dot_general(
        q_ref[...], c_ref[...],
        dimension_numbers=(((1,), (1,)), ((), ())),
        preferred_element_type=jnp.float32,
    )


def _full_topk(queries, candidates, identifiers, kk):
    nq, d = queries.shape
    n, _ = candidates.shape
    blk = 8192
    scores = pl.pallas_call(
        _mm_kernel,
        grid=(pl.cdiv(n, blk),),
        in_specs=[
            pl.BlockSpec((nq, d), lambda j: (0, 0)),
            pl.BlockSpec((blk, d), lambda j: (j, 0)),
        ],
        out_specs=pl.BlockSpec((nq, blk), lambda j: (0, j)),
        out_shape=jax.ShapeDtypeStruct((nq, n), jnp.float32),
    )(queries, candidates)
    values, indices = jax.lax.top_k(scores, kk)
    return values, jnp.take(identifiers, indices, axis=0)


def kernel(queries, candidates, identifiers, k):
    nq, d = queries.shape
    n, _ = candidates.shape
    kk = 10

    ct = candidates.T.astype(jnp.bfloat16)
    shape_f = jax.ShapeDtypeStruct((_NQ, _NCLS), jnp.float32)
    shape_i = jax.ShapeDtypeStruct((_NQ, _NCLS), jnp.int32)
    acc_spec = pl.BlockSpec((_NQ, _NCLS), lambda j: (0, 0))
    v3m_spec = pl.BlockSpec((_NQ, _LANES), lambda j: (0, 0))
    shape_v3m = jax.ShapeDtypeStruct((_NQ, _LANES), jnp.float32)
    v1, i1, v2, i2, v3 = pl.pallas_call(
        functools.partial(_stream_kernel, n),
        grid=(pl.cdiv(n, _BLK),),
        in_specs=[
            pl.BlockSpec((nq, d), lambda j: (0, 0)),
            pl.BlockSpec((d, _BLK), lambda j: (0, j)),
        ],
        out_specs=(acc_spec,) * 4 + (v3m_spec,),
        out_shape=(shape_f, shape_i, shape_f, shape_i, shape_v3m),
    )(queries, ct)

    valso, idxo, flago = pl.pallas_call(
        _extract_kernel,
        in_specs=[pl.BlockSpec((_NQ, _NCLS), lambda: (0, 0))] * 4
        + [pl.BlockSpec((_NQ, _LANES), lambda: (0, 0))],
        out_specs=(pl.BlockSpec((_NQ, _LANES), lambda: (0, 0)),) * 3,
        out_shape=(jax.ShapeDtypeStruct((_NQ, _LANES), jnp.float32),
                   jax.ShapeDtypeStruct((_NQ, _LANES), jnp.int32),
                   jax.ShapeDtypeStruct((_NQ, _LANES), jnp.float32)),
    )(v1, i1, v2, i2, v3)
    vals = valso[:, :kk]
    idx = idxo[:, :kk]
    need_fallback = jnp.any(flago[:, 0] > 0)

    fast = (vals, jnp.take(identifiers, idx, axis=0))
    return jax.lax.cond(
        need_fallback,
        lambda: _full_topk(queries, candidates, identifiers, kk),
        lambda: fast,
    )


# speedup vs baseline: 1.0148x; 1.0148x over previous
import functools

import jax
import jax.numpy as jnp
from jax.experimental import pallas as pl

_NQ = 64
_BLK = 8192
_NCLS = 2048
_LANES = 128
_NEG = float("-inf")


def _stream_kernel(n, q_ref, c_ref, v1_ref, i1_ref, v2_ref, i2_ref, v3m_ref):
    j = pl.program_id(0)

    @pl.when(j == 0)
    def _init():
        v1_ref[...] = jnp.full((_NQ, _NCLS), _NEG, jnp.float32)
        v2_ref[...] = jnp.full((_NQ, _NCLS), _NEG, jnp.float32)
        v3m_ref[...] = jnp.full((_NQ, _LANES), _NEG, jnp.float32)

    scores = jax.lax.dot_general(
        q_ref[...].astype(jnp.bfloat16), c_ref[...],
        dimension_numbers=(((1,), (0,)), ((), ())),
        preferred_element_type=jnp.float32,
    )
    lane = jax.lax.broadcasted_iota(jnp.int32, (_NQ, _LANES), 1)
    base = j * _BLK
    quart = _BLK // 4
    v3m = v3m_ref[...]

    def _pair(xv, xi, yv, yi):
        c = yv > xv
        return (jnp.where(c, yv, xv), jnp.where(c, yi, xi),
                jnp.where(c, xv, yv), jnp.where(c, xi, yi), c)

    for g in range(_NCLS // _LANES):
        sl = slice(g * _LANES, (g + 1) * _LANES)
        col = g * _LANES
        xs, xis = [], []
        for qq in range(4):
            mi = lane + (base + qq * quart + col)
            v = scores[:, qq * quart + col:qq * quart + col + _LANES]
            xs.append(jnp.where(mi < n, v, _NEG))
            xis.append(mi)

        h1, h1i, l1, l1i, _ = _pair(xs[0], xis[0], xs[1], xis[1])
        h2, h2i, l2, l2i, _ = _pair(xs[2], xis[2], xs[3], xis[3])
        c = h2 > h1
        hi = jnp.where(c, h2, h1)
        hi_i = jnp.where(c, h2i, h1i)
        m1 = jnp.minimum(h1, h2)
        m1i = jnp.where(c, h1i, h2i)
        s = jnp.where(c, l2, l1)
        s_i = jnp.where(c, l2i, l1i)
        cq = s > m1
        lo = jnp.where(cq, s, m1)
        lo_i = jnp.where(cq, s_i, m1i)
        rest = jnp.maximum(jnp.where(cq, m1, s), jnp.where(c, l1, l2))

        v1 = v1_ref[:, sl]
        i1 = i1_ref[:, sl]
        v2 = v2_ref[:, sl]
        i2 = i2_ref[:, sl]

        c1 = hi > v1
        v1_ref[:, sl] = jnp.where(c1, hi, v1)
        i1_ref[:, sl] = jnp.where(c1, hi_i, i1)
        m2 = jnp.minimum(v1, hi)
        m2i = jnp.where(c1, i1, hi_i)
        s2 = jnp.where(c1, lo, v2)
        s2i = jnp.where(c1, lo_i, i2)
        c2 = s2 > m2
        v2_ref[:, sl] = jnp.where(c2, s2, m2)
        i2_ref[:, sl] = jnp.where(c2, s2i, m2i)

        t1 = jnp.where(c2, m2, s2)
        t2 = jnp.where(c1, v2, lo)
        v3m = jnp.maximum(v3m, jnp.maximum(jnp.maximum(t1, t2), rest))

    v3m_ref[...] = v3m


_IMAX = 2**31 - 1


def _extract_kernel(v1_ref, i1_ref, v2_ref, i2_ref, v3_ref,
                    vals_ref, idx_ref, flag_ref):
    v1 = v1_ref[...]
    v2 = v2_ref[...]
    i1 = i1_ref[...]
    i2 = i2_ref[...]
    vals_ref[...] = jnp.zeros((_NQ, _LANES), jnp.float32)
    idx_ref[...] = jnp.zeros((_NQ, _LANES), jnp.int32)
    cm = None
    for t in range(10):
        m = jnp.maximum(jnp.max(v1, axis=1, keepdims=True),
                        jnp.max(v2, axis=1, keepdims=True))
        eq1 = v1 == m
        eq2 = v2 == m
        ix = jnp.minimum(
            jnp.min(jnp.where(eq1, i1, _IMAX), axis=1, keepdims=True),
            jnp.min(jnp.where(eq2, i2, _IMAX), axis=1, keepdims=True))
        v1 = jnp.where(eq1 & (i1 == ix), _NEG, v1)
        v2 = jnp.where(eq2 & (i2 == ix), _NEG, v2)
        vals_ref[:, t:t + 1] = m
        idx_ref[:, t:t + 1] = ix
        cm = m
    f1 = jnp.max(jnp.where(v3_ref[...] >= cm, 1.0, 0.0), axis=1, keepdims=True)
    flag_ref[...] = jnp.zeros((_NQ, _LANES), jnp.float32)
    flag_ref[:, 0:1] = f1


def _mm_kernel(q_ref, c_ref, o_ref):
    o_ref[...] = jax.lax.dot_general(
        q_ref[...], c_ref[...],
        dimension_numbers=(((1,), (1,)), ((), ())),
        preferred_element_type=jnp.float32,
    )


def _full_topk(queries, candidates, identifiers, kk):
    nq, d = queries.shape
    n, _ = candidates.shape
    blk = 8192
    scores = pl.pallas_call(
        _mm_kernel,
        grid=(pl.cdiv(n, blk),),
        in_specs=[
            pl.BlockSpec((nq, d), lambda j: (0, 0)),
            pl.BlockSpec((blk, d), lambda j: (j, 0)),
        ],
        out_specs=pl.BlockSpec((nq, blk), lambda j: (0, j)),
        out_shape=jax.ShapeDtypeStruct((nq, n), jnp.float32),
    )(queries, candidates)
    values, indices = jax.lax.top_k(scores, kk)
    return values, jnp.take(identifiers, indices, axis=0)


def kernel(queries, candidates, identifiers, k):
    nq, d = queries.shape
    n, _ = candidates.shape
    kk = 10

    ct = candidates.T.astype(jnp.bfloat16)
    shape_f = jax.ShapeDtypeStruct((_NQ, _NCLS), jnp.float32)
    shape_i = jax.ShapeDtypeStruct((_NQ, _NCLS), jnp.int32)
    acc_spec = pl.BlockSpec((_NQ, _NCLS), lambda j: (0, 0))
    v3m_spec = pl.BlockSpec((_NQ, _LANES), lambda j: (0, 0))
    shape_v3m = jax.ShapeDtypeStruct((_NQ, _LANES), jnp.float32)
    v1, i1, v2, i2, v3 = pl.pallas_call(
        functools.partial(_stream_kernel, n),
        grid=(pl.cdiv(n, _BLK),),
        in_specs=[
            pl.BlockSpec((nq, d), lambda j: (0, 0)),
            pl.BlockSpec((d, _BLK), lambda j: (0, j)),
        ],
        out_specs=(acc_spec,) * 4 + (v3m_spec,),
        out_shape=(shape_f, shape_i, shape_f, shape_i, shape_v3m),
    )(queries, ct)

    valso, idxo, flago = pl.pallas_call(
        _extract_kernel,
        in_specs=[pl.BlockSpec((_NQ, _NCLS), lambda: (0, 0))] * 4
        + [pl.BlockSpec((_NQ, _LANES), lambda: (0, 0))],
        out_specs=(pl.BlockSpec((_NQ, _LANES), lambda: (0, 0)),) * 3,
        out_shape=(jax.ShapeDtypeStruct((_NQ, _LANES), jnp.float32),
                   jax.ShapeDtypeStruct((_NQ, _LANES), jnp.int32),
                   jax.ShapeDtypeStruct((_NQ, _LANES), jnp.float32)),
    )(v1, i1, v2, i2, v3)
    vals = valso[:, :kk]
    idx = idxo[:, :kk]
    need_fallback = jnp.any(flago[:, 0] > 0)

    fast = (vals, jnp.take(identifiers, idx, axis=0))
    return jax.lax.cond(
        need_fallback,
        lambda: _full_topk(queries, candidates, identifiers, kk),
        lambda: fast,
    )
